# baseline (speedup 1.0000x reference)
import jax
import jax.numpy as jnp
from jax.experimental import pallas as pl

_BM = 2000


def _mlp_block(x_ref, w1t_ref, b1_ref, wot_ref, bo_ref, out_ref):
    h = jnp.dot(x_ref[...], w1t_ref[...], preferred_element_type=jnp.float32)
    h = jnp.maximum(h + b1_ref[...], 0.0)
    out = jnp.dot(h, wot_ref[...], preferred_element_type=jnp.float32)
    out_ref[...] = out + bo_ref[...]


def kernel(x, W1, b1, W_out, b_out):
    n, d = x.shape
    hdim = W1.shape[0]
    c = W_out.shape[0]
    w1t = W1.T
    wot = W_out.T
    b1r = b1.reshape(1, hdim)
    bor = b_out.reshape(1, c)
    grid = (pl.cdiv(n, _BM),)
    return pl.pallas_call(
        _mlp_block,
        grid=grid,
        in_specs=[
            pl.BlockSpec((_BM, d), lambda i: (i, 0)),
            pl.BlockSpec((d, hdim), lambda i: (0, 0)),
            pl.BlockSpec((1, hdim), lambda i: (0, 0)),
            pl.BlockSpec((hdim, c), lambda i: (0, 0)),
            pl.BlockSpec((1, c), lambda i: (0, 0)),
        ],
        out_specs=pl.BlockSpec((_BM, c), lambda i: (i, 0)),
        out_shape=jax.ShapeDtypeStruct((n, c), jnp.float32),
    )(x, w1t, b1r, wot, bor)
